# baseline (speedup 1.0000x reference)
import functools

import jax
import jax.numpy as jnp
from jax import lax
from jax.experimental import pallas as pl
from jax.experimental.pallas import tpu as pltpu
from jax.experimental.pallas import tpu_sc as plsc

_NC = 2
_NS = 16
_NW = _NC * _NS
_CHUNK = 128
_IB = 5
_NSLOT = 3


def _make_gather(n_chunks, n_vocab, d, n_rounds):
    mesh = plsc.VectorSubcoreMesh(core_axis_name="c", subcore_axis_name="s")
    assert n_rounds % _IB == 0 and n_rounds >= 2 * _IB
    assert (n_rounds - 2) % _NSLOT == 0
    n_blocks = n_rounds // _IB
    v_step = (n_vocab // _NS) // 8 * 8
    v_size = n_vocab - v_step * (_NS - 1)
    assert v_size >= 8 and v_step * (_NS - 1) + v_size <= n_vocab

    @functools.partial(
        pl.kernel,
        mesh=mesh,
        out_type=jax.ShapeDtypeStruct((n_chunks, _CHUNK, d), jnp.float32),
        scratch_types=[
            pltpu.VMEM_SHARED((n_vocab, d), jnp.float32),
            pltpu.VMEM((2 * _IB, 1, _CHUNK), jnp.int32),
            pltpu.VMEM((_NSLOT, _CHUNK, d), jnp.float32),
            pltpu.SemaphoreType.DMA,
            pltpu.SemaphoreType.DMA,
            pltpu.SemaphoreType.DMA,
            pltpu.SemaphoreType.DMA,
            pltpu.SemaphoreType.DMA,
            pltpu.SemaphoreType.DMA,
        ],
    )
    def k(idx_hbm, table_hbm, out_hbm, tab_sh, idx_v, rows_v,
          gsem, ssem0, ssem1, ssem2, isem, tsem):
        sid = lax.axis_index("s")
        wid = sid * _NC + lax.axis_index("c")

        def t_copy():
            return pltpu.make_async_copy(
                table_hbm.at[pl.ds(sid * v_step, v_size)],
                tab_sh.at[pl.ds(sid * v_step, v_size)], tsem)

        t_copy().start()
        ibase = wid * n_rounds
        cbase = wid * n_rounds
        ssems = (ssem0, ssem1, ssem2)

        def i_copy(blk):
            return pltpu.make_async_copy(
                idx_hbm.at[pl.ds(ibase + blk * _IB, _IB)],
                idx_v.at[pl.ds((blk % 2) * _IB, _IB)], isem)

        def g_copy(r, s, src):
            return pltpu.make_async_copy(
                src.at[idx_v.at[r % (2 * _IB), 0]], rows_v.at[s], gsem)

        def s_copy(r, s):
            return pltpu.make_async_copy(
                rows_v.at[s], out_hbm.at[cbase + r], ssems[s])

        pltpu.sync_copy(idx_hbm.at[pl.ds(ibase, _IB)],
                        idx_v.at[pl.ds(0, _IB)])
        i_copy(1).start()

        def round_body(r):
            blk = r // _IB

            @pl.when(jnp.logical_and(r % _IB == 0,
                                     jnp.logical_and(r > 0,
                                                     blk < n_blocks - 1)))
            def _():
                i_copy(blk + 1).start()

            @pl.when(jnp.logical_and(r % _IB == _IB - 2,
                                     r + 2 < n_rounds))
            def _():
                i_copy((r + 2) // _IB).wait()

        g_copy(0, 0, table_hbm).start()
        g_copy(1, 1, table_hbm).start()
        g_copy(0, 0, table_hbm).wait()
        s_copy(0, 0).start()
        g_copy(2, 2, table_hbm).start()

        _E = 13

        def body_a(i, carry):
            for u in range(_NSLOT):
                r = 1 + _NSLOT * i + u
                s = (1 + u) % _NSLOT
                round_body(r)
                g_copy(r, s, table_hbm).wait()
                s_copy(r, s).start()
                s_copy(r - 1, u).wait()

                @pl.when(r + 2 < _E)
                def _():
                    g_copy(r + 2, u, table_hbm).start()
            return carry

        def body_b(i, carry):
            for u in range(_NSLOT):
                r = 1 + _NSLOT * i + u
                s = (1 + u) % _NSLOT
                round_body(r)
                g_copy(r, s, tab_sh).wait()
                s_copy(r, s).start()

                @pl.when(r + 2 < n_rounds)
                def _():
                    s_copy(r - 1, u).wait()
                    g_copy(r + 2, u, tab_sh).start()
            return carry

        n_a = (_E - 1) // _NSLOT
        lax.fori_loop(0, n_a, body_a, 0)
        t_copy().wait()
        plsc.subcore_barrier()
        g_copy(_E, _E % _NSLOT, tab_sh).start()
        g_copy(_E + 1, (_E + 1) % _NSLOT, tab_sh).start()
        lax.fori_loop(n_a, (n_rounds - 2) // _NSLOT, body_b, 0)

        rl = n_rounds - 1
        g_copy(rl, rl % _NSLOT, tab_sh).wait()
        s_copy(rl, rl % _NSLOT).start()
        for r in range(n_rounds - 3, n_rounds):
            s_copy(r, r % _NSLOT).wait()

    return k


def kernel(x, table):
    b, s = x.shape
    n_vocab, d = table.shape
    n = b * s
    n_chunks = n // _CHUNK
    n_rounds = n_chunks // _NW
    idx = x.reshape(n_chunks, 1, _CHUNK).astype(jnp.int32)
    out = _make_gather(n_chunks, n_vocab, d, n_rounds)(idx, table)
    return out.reshape(b, s, d)


# speedup vs baseline: 1.0357x; 1.0357x over previous
import functools

import jax
import jax.numpy as jnp
from jax import lax
from jax.experimental import pallas as pl
from jax.experimental.pallas import tpu as pltpu
from jax.experimental.pallas import tpu_sc as plsc

_NC = 2
_NS = 16
_NW = _NC * _NS
_CHUNK = 128
_IB = 5
_NSLOT = 3


def _make_gather(n_chunks, n_vocab, d, n_rounds):
    mesh = plsc.VectorSubcoreMesh(core_axis_name="c", subcore_axis_name="s")
    assert n_rounds % _IB == 0 and n_rounds >= 2 * _IB
    assert (n_rounds - 2) % _NSLOT == 0
    n_blocks = n_rounds // _IB
    v_step = (n_vocab // _NS) // 8 * 8
    v_size = n_vocab - v_step * (_NS - 1)
    assert v_size >= 8 and v_step * (_NS - 1) + v_size <= n_vocab

    @functools.partial(
        pl.kernel,
        mesh=mesh,
        out_type=jax.ShapeDtypeStruct((n_chunks, _CHUNK, d), jnp.float32),
        scratch_types=[
            pltpu.VMEM_SHARED((n_vocab, d), jnp.float32),
            pltpu.VMEM((2 * _IB, 1, _CHUNK), jnp.int32),
            pltpu.VMEM((_NSLOT, _CHUNK, d), jnp.float32),
            pltpu.SemaphoreType.DMA,
            pltpu.SemaphoreType.DMA,
            pltpu.SemaphoreType.DMA,
            pltpu.SemaphoreType.DMA,
            pltpu.SemaphoreType.DMA,
        ],
    )
    def k(idx_hbm, table_hbm, out_hbm, tab_sh, idx_v, rows_v,
          gsem, ssem0, ssem1, ssem2, isem):
        sid = lax.axis_index("s")
        wid = sid * _NC + lax.axis_index("c")
        pltpu.sync_copy(table_hbm.at[pl.ds(sid * v_step, v_size)],
                        tab_sh.at[pl.ds(sid * v_step, v_size)])
        ibase = wid * n_rounds
        cbase = wid * n_rounds
        ssems = (ssem0, ssem1, ssem2)

        def i_copy(blk):
            return pltpu.make_async_copy(
                idx_hbm.at[pl.ds(ibase + blk * _IB, _IB)],
                idx_v.at[pl.ds((blk % 2) * _IB, _IB)], isem)

        def g_copy(r, s):
            return pltpu.make_async_copy(
                tab_sh.at[idx_v.at[r % (2 * _IB), 0]], rows_v.at[s], gsem)

        def s_copy(r, s):
            return pltpu.make_async_copy(
                rows_v.at[s], out_hbm.at[cbase + r], ssems[s])

        pltpu.sync_copy(idx_hbm.at[pl.ds(ibase, _IB)],
                        idx_v.at[pl.ds(0, _IB)])
        i_copy(1).start()
        plsc.subcore_barrier()

        def round_body(r):
            blk = r // _IB

            @pl.when(jnp.logical_and(r % _IB == 0,
                                     jnp.logical_and(r > 0,
                                                     blk < n_blocks - 1)))
            def _():
                i_copy(blk + 1).start()

            @pl.when(jnp.logical_and(r % _IB == _IB - 2,
                                     r + 2 < n_rounds))
            def _():
                i_copy((r + 2) // _IB).wait()

        g_copy(0, 0).start()
        g_copy(1, 1).start()
        g_copy(0, 0).wait()
        s_copy(0, 0).start()
        g_copy(2, 2).start()

        def body(i, carry):
            for u in range(_NSLOT):
                r = 1 + _NSLOT * i + u
                s = (1 + u) % _NSLOT
                round_body(r)
                g_copy(r, s).wait()
                s_copy(r, s).start()

                @pl.when(r + 2 < n_rounds)
                def _():
                    s_copy(r - 1, u).wait()
                    g_copy(r + 2, u).start()
            return carry

        lax.fori_loop(0, (n_rounds - 2) // _NSLOT, body, 0)
        rl = n_rounds - 1
        g_copy(rl, rl % _NSLOT).wait()
        s_copy(rl, rl % _NSLOT).start()
        for r in range(n_rounds - 3, n_rounds):
            s_copy(r, r % _NSLOT).wait()

    return k


def kernel(x, table):
    b, s = x.shape
    n_vocab, d = table.shape
    n = b * s
    n_chunks = n // _CHUNK
    n_rounds = n_chunks // _NW
    idx = x.reshape(n_chunks, 1, _CHUNK).astype(jnp.int32)
    out = _make_gather(n_chunks, n_vocab, d, n_rounds)(idx, table)
    return out.reshape(b, s, d)
